# baseline (speedup 1.0000x reference)
import functools

import jax
import jax.numpy as jnp
from jax import lax
from jax.experimental import pallas as pl
from jax.experimental.pallas import tpu as pltpu
from jax.experimental.pallas import tpu_sc as plsc

_NW = 32
_ECH = 128
_K = 8


def _proj4_body(x_ref, w_ref, b_ref, o_ref):
    bm = x_ref.shape[0]
    n = w_ref.shape[1]
    del bm
    o_ref[...] = jnp.maximum(
        jnp.dot(x_ref[...], w_ref[...], preferred_element_type=jnp.float32)
        + b_ref[...],
        0.0,
    )


def _relu_proj4(x, w, b, bm):
    m, k = x.shape
    n = w.shape[1]
    grid = (m // bm,)
    return pl.pallas_call(
        _proj4_body,
        grid=grid,
        in_specs=[
            pl.BlockSpec((bm, k), lambda i: (i, 0)),
            pl.BlockSpec((k, n), lambda i: (0, 0)),
            pl.BlockSpec((1, n), lambda i: (0, 0)),
        ],
        out_specs=pl.BlockSpec((bm, n), lambda i: (i, 0)),
        out_shape=jax.ShapeDtypeStruct((m, n), jnp.float32),
    )(x, w, b.reshape(1, n))


def _pre_edge_body(x_ref, w_ref, b_ref, o_ref):
    o_ref[...] = jnp.maximum(
        jnp.dot(x_ref[...], w_ref[...], preferred_element_type=jnp.float32)
        + b_ref[...],
        0.0,
    ).astype(jnp.bfloat16)


def _pre_edge(x, w, b, bm):
    m, k = x.shape
    n = w.shape[1]
    grid = (m // bm,)
    return pl.pallas_call(
        _pre_edge_body,
        grid=grid,
        in_specs=[
            pl.BlockSpec((bm, k), lambda i: (i, 0)),
            pl.BlockSpec((k, n), lambda i: (0, 0)),
            pl.BlockSpec((1, n), lambda i: (0, 0)),
        ],
        out_specs=pl.BlockSpec((bm, n), lambda i: (i, 0)),
        out_shape=jax.ShapeDtypeStruct((m, n), jnp.bfloat16),
    )(x, w, b.reshape(1, n))


def _bilinear_body(t_ref, u_ref, w2_ref, b2r_ref, r_ref, s_ref, o_ref):
    u = u_ref[...].astype(jnp.bfloat16)
    g = jnp.dot(t_ref[...], w2_ref[...], preferred_element_type=jnp.float32)
    urep = jnp.dot(u, r_ref[...], preferred_element_type=jnp.float32)
    prod = (g * urep).astype(jnp.bfloat16)
    o_ref[...] = jnp.dot(
        prod, s_ref[...], preferred_element_type=jnp.float32
    ) + jnp.dot(u_ref[...], b2r_ref[...], preferred_element_type=jnp.float32)


def _bilinear(t, u, we2, b2r, rmat, smat, be):
    e, eh = t.shape
    hh = we2.shape[1]
    h = rmat.shape[0]
    grid = (e // be,)
    return pl.pallas_call(
        _bilinear_body,
        grid=grid,
        in_specs=[
            pl.BlockSpec((be, eh), lambda i: (i, 0)),
            pl.BlockSpec((be, h), lambda i: (i, 0)),
            pl.BlockSpec((eh, hh), lambda i: (0, 0)),
            pl.BlockSpec((h, h), lambda i: (0, 0)),
            pl.BlockSpec((h, hh), lambda i: (0, 0)),
            pl.BlockSpec((hh, h), lambda i: (0, 0)),
        ],
        out_specs=pl.BlockSpec((be, h), lambda i: (i, 0)),
        out_shape=jax.ShapeDtypeStruct((e, h), jnp.float32),
    )(t, u, we2, b2r, rmat, smat)


def _gru_body(p0_ref, p1_ref, bc_ref, hid_ref, wih_ref, whh_ref, bih_ref,
              bhh_ref, o_ref):
    h = bc_ref.shape[1]
    agg = p0_ref[...] + p1_ref[...] + bc_ref[...]
    hid = hid_ref[...]
    m = jnp.maximum(agg, 0.0)
    gi = (
        jnp.dot(m, wih_ref[...], preferred_element_type=jnp.float32)
        + bih_ref[...]
    )
    gh = (
        jnp.dot(hid, whh_ref[...], preferred_element_type=jnp.float32)
        + bhh_ref[...]
    )
    r = jax.nn.sigmoid(gi[:, :h] + gh[:, :h])
    z = jax.nn.sigmoid(gi[:, h : 2 * h] + gh[:, h : 2 * h])
    n = jnp.tanh(gi[:, 2 * h :] + r * gh[:, 2 * h :])
    o_ref[...] = (1.0 - z) * n + z * hid


def _gru(p0, p1, b_conv, hid, wih_t, whh_t, bih, bhh, bn):
    h = b_conv.shape[0]
    nn = hid.shape[0]
    grid = (nn // bn,)
    return pl.pallas_call(
        _gru_body,
        grid=grid,
        in_specs=[
            pl.BlockSpec((bn, h), lambda i: (i, 0)),
            pl.BlockSpec((bn, h), lambda i: (i, 0)),
            pl.BlockSpec((1, h), lambda i: (0, 0)),
            pl.BlockSpec((bn, h), lambda i: (i, 0)),
            pl.BlockSpec((h, 3 * h), lambda i: (0, 0)),
            pl.BlockSpec((h, 3 * h), lambda i: (0, 0)),
            pl.BlockSpec((1, 3 * h), lambda i: (0, 0)),
            pl.BlockSpec((1, 3 * h), lambda i: (0, 0)),
        ],
        out_specs=pl.BlockSpec((bn, h), lambda i: (i, 0)),
        out_shape=jax.ShapeDtypeStruct((nn, h), jnp.float32),
    )(
        p0,
        p1,
        b_conv.reshape(1, h),
        hid,
        wih_t,
        whh_t,
        bih.reshape(1, 3 * h),
        bhh.reshape(1, 3 * h),
    )


def _s2s_body(
    h_ref,
    wih0_ref, whh0_ref, bih0_ref, bhh0_ref,
    wih1_ref, whh1_ref, bih1_ref, bhh1_ref,
    wih2_ref, whh2_ref, bih2_ref, bhh2_ref,
    wd1_ref, bd1_ref, wd2_ref, bd2_ref,
    o_ref,
):
    hdim = h_ref.shape[1]
    hmat = h_ref[...]
    wih = [wih0_ref[...], wih1_ref[...], wih2_ref[...]]
    whh = [whh0_ref[...], whh1_ref[...], whh2_ref[...]]
    bih = [bih0_ref[...], bih1_ref[...], bih2_ref[...]]
    bhh = [bhh0_ref[...], bhh1_ref[...], bhh2_ref[...]]
    q_star = jnp.zeros((1, 2 * hdim), dtype=jnp.float32)
    hs = [jnp.zeros((1, hdim), dtype=jnp.float32) for _ in range(3)]
    cs = [jnp.zeros((1, hdim), dtype=jnp.float32) for _ in range(3)]
    for _ in range(6):
        x = q_star
        for l in range(3):
            g = (
                jnp.dot(x, wih[l], preferred_element_type=jnp.float32)
                + bih[l]
                + jnp.dot(hs[l], whh[l], preferred_element_type=jnp.float32)
                + bhh[l]
            )
            i = jax.nn.sigmoid(g[:, :hdim])
            f = jax.nn.sigmoid(g[:, hdim : 2 * hdim])
            gg = jnp.tanh(g[:, 2 * hdim : 3 * hdim])
            o = jax.nn.sigmoid(g[:, 3 * hdim :])
            cs[l] = f * cs[l] + i * gg
            hs[l] = o * jnp.tanh(cs[l])
            x = hs[l]
        q = x
        e = jnp.dot(hmat, q.reshape(hdim, 1), preferred_element_type=jnp.float32)
        emax = jnp.max(e)
        a = jnp.exp(e - emax)
        denom = jnp.sum(a)
        readout = jnp.sum(a * hmat, axis=0, keepdims=True) / denom
        q_star = jnp.concatenate([q, readout], axis=1)
    out = (
        jnp.dot(
            jnp.maximum(
                jnp.dot(q_star, wd1_ref[...], preferred_element_type=jnp.float32)
                + bd1_ref[...],
                0.0,
            ),
            wd2_ref[...],
            preferred_element_type=jnp.float32,
        )
        + bd2_ref[...]
    )
    o_ref[...] = out


def _set2set_decode(hmat, lstm_t, wd1, bd1, wd2, bd2):
    ph = wd1.shape[1]
    tasks = wd2.shape[1]
    args = [hmat]
    in_specs = [pl.BlockSpec(hmat.shape, lambda: (0, 0))]
    for (wih_t, whh_t, bih, bhh) in lstm_t:
        for a in (wih_t, whh_t, bih.reshape(1, -1), bhh.reshape(1, -1)):
            args.append(a)
            in_specs.append(pl.BlockSpec(a.shape, lambda: (0, 0)))
    for a in (wd1, bd1.reshape(1, ph), wd2, bd2.reshape(1, tasks)):
        args.append(a)
        in_specs.append(pl.BlockSpec(a.shape, lambda: (0, 0)))
    return pl.pallas_call(
        _s2s_body,
        in_specs=in_specs,
        out_specs=pl.BlockSpec((1, tasks), lambda: (0, 0)),
        out_shape=jax.ShapeDtypeStruct((1, tasks), jnp.float32),
    )(*args)


def _sc_gather(h, srcm):
    nn, d = h.shape
    nch = srcm.shape[0]
    etot = nch * _ECH
    per = nch // _NW
    rem = nch % _NW
    mesh = plsc.VectorSubcoreMesh(core_axis_name="c", subcore_axis_name="s")

    @functools.partial(
        pl.kernel,
        mesh=mesh,
        out_type=jax.ShapeDtypeStruct((etot, d), jnp.float32),
        scratch_types=[
            pltpu.VMEM((_K, _ECH), jnp.int32),
            pltpu.VMEM((_K * _ECH, d), jnp.float32),
            pltpu.SemaphoreType.DMA,
        ],
        compiler_params=pltpu.CompilerParams(use_tc_tiling_on_sc=False),
    )
    def k(h_hbm, srcm_hbm, out_hbm, idx2, rows_v, sem):
        c = lax.axis_index("c")
        s = lax.axis_index("s")
        wid = s * 2 + c
        base_row = wid * per

        def outer(row0, nrows):
            pltpu.sync_copy(srcm_hbm.at[pl.ds(row0, nrows)], idx2.at[pl.ds(0, nrows)])
            handles = []
            for j in range(nrows):
                handles.append(
                    pltpu.async_copy(
                        h_hbm.at[idx2.at[j]],
                        rows_v.at[pl.ds(j * _ECH, _ECH)],
                        sem,
                    )
                )
            for hd in handles:
                hd.wait()
            pltpu.sync_copy(
                rows_v.at[pl.ds(0, nrows * _ECH)],
                out_hbm.at[pl.ds(row0 * _ECH, nrows * _ECH)],
            )

        nfull, tail = divmod(per, _K)
        for o in range(nfull):
            outer(base_row + o * _K, _K)
        if tail:
            outer(base_row + nfull * _K, tail)
        if rem:
            @pl.when(wid < rem)
            def _():
                outer(_NW * per + wid, 1)

    return k(h, srcm)


def _sc_scatter(msg, dstm, zeros_nd):
    nch = dstm.shape[0]
    nn, d = zeros_nd.shape
    per = nch // _NW
    rem = nch % _NW
    rpt = nn // 16
    mesh = plsc.VectorSubcoreMesh(core_axis_name="c", subcore_axis_name="s")

    @functools.partial(
        pl.kernel,
        mesh=mesh,
        out_type=jax.ShapeDtypeStruct((2, nn, d), jnp.float32),
        scratch_types=[
            pltpu.VMEM((_K, _ECH), jnp.int32),
            pltpu.VMEM((_K * _ECH, d), jnp.float32),
            pltpu.VMEM_SHARED((nn, d), jnp.float32),
            pltpu.SemaphoreType.DMA,
        ],
        compiler_params=pltpu.CompilerParams(use_tc_tiling_on_sc=False),
    )
    def k(msg_hbm, dstm_hbm, zeros_hbm, out_hbm, idx2, rows_v, acc_sh, sem):
        c = lax.axis_index("c")
        s = lax.axis_index("s")
        wid = s * 2 + c
        base_row = wid * per
        msg_r = msg_hbm
        pltpu.sync_copy(
            zeros_hbm.at[pl.ds(s * rpt, rpt)], acc_sh.at[pl.ds(s * rpt, rpt)]
        )
        plsc.subcore_barrier()

        def outer(row0, nrows):
            pltpu.sync_copy(dstm_hbm.at[pl.ds(row0, nrows)], idx2.at[pl.ds(0, nrows)])
            pltpu.sync_copy(
                msg_r.at[pl.ds(row0 * _ECH, nrows * _ECH)],
                rows_v.at[pl.ds(0, nrows * _ECH)],
            )
            handles = []
            for j in range(nrows):
                handles.append(
                    pltpu.async_copy(
                        rows_v.at[pl.ds(j * _ECH, _ECH)],
                        acc_sh.at[idx2.at[j]],
                        sem,
                        add=True,
                    )
                )
            for hd in handles:
                hd.wait()

        nfull, tail = divmod(per, _K)
        for o in range(nfull):
            outer(base_row + o * _K, _K)
        if tail:
            outer(base_row + nfull * _K, tail)
        if rem:
            @pl.when(wid < rem)
            def _():
                outer(_NW * per + wid, 1)

        plsc.subcore_barrier()
        pltpu.sync_copy(
            acc_sh.at[pl.ds(s * rpt, rpt)], out_hbm.at[c, pl.ds(s * rpt, rpt)]
        )

    return k(msg, dstm, zeros_nd)


def kernel(node_feats, edge_feats, edge_index, W_proj, b_proj, We1, be1, We2,
           be2, b_conv, gWih, gWhh, gbih, gbhh, lWih0, lWhh0, lbih0, lbhh0,
           lWih1, lWhh1, lbih1, lbhh1, lWih2, lWhh2, lbih2, lbhh2, Wd1, bd1,
           Wd2, bd2):
    nn = node_feats.shape[0]
    etot = edge_feats.shape[0]
    h = W_proj.shape[1]
    hh = We2.shape[1]
    srcm = edge_index[0].reshape(etot // _ECH, _ECH)
    dstm = edge_index[1].reshape(etot // _ECH, _ECH)

    cols = jnp.arange(hh, dtype=jnp.int32)
    rmat = (cols[None, :] // h == jnp.arange(h, dtype=jnp.int32)[:, None]).astype(
        jnp.bfloat16
    )
    smat = (cols[:, None] % h == jnp.arange(h, dtype=jnp.int32)[None, :]).astype(
        jnp.bfloat16
    )
    zeros_nd = jnp.zeros((nn, h), dtype=jnp.float32)
    we2_b = We2.astype(jnp.bfloat16)
    b2r = be2.reshape(h, h)

    t = _pre_edge(edge_feats, We1, be1, 2000)
    hcur = _relu_proj4(node_feats, W_proj, b_proj, 10000)
    hidden = hcur

    wih_t = gWih.T
    whh_t = gWhh.T

    for _ in range(5):
        u = _sc_gather(hcur, srcm)
        msg = _bilinear(t, u, we2_b, b2r, rmat, smat, 1600)
        partials = _sc_scatter(msg, dstm, zeros_nd)
        hidden = _gru(partials[0], partials[1], b_conv, hidden, wih_t, whh_t,
                      gbih, gbhh, 10000)
        hcur = hidden

    lstm_t = [
        (lWih0.T, lWhh0.T, lbih0, lbhh0),
        (lWih1.T, lWhh1.T, lbih1, lbhh1),
        (lWih2.T, lWhh2.T, lbih2, lbhh2),
    ]
    return _set2set_decode(hcur, lstm_t, Wd1, bd1, Wd2, bd2)


# speedup vs baseline: 1.1848x; 1.1848x over previous
import functools

import jax
import jax.numpy as jnp
from jax import lax
from jax.experimental import pallas as pl
from jax.experimental.pallas import tpu as pltpu
from jax.experimental.pallas import tpu_sc as plsc

_NW = 32
_ECH = 128
_K = 8


def _proj4_body(x_ref, w_ref, b_ref, o_ref):
    bm = x_ref.shape[0]
    n = w_ref.shape[1]
    del bm
    o_ref[...] = jnp.maximum(
        jnp.dot(x_ref[...], w_ref[...], preferred_element_type=jnp.float32)
        + b_ref[...],
        0.0,
    )


def _relu_proj4(x, w, b, bm):
    m, k = x.shape
    n = w.shape[1]
    grid = (m // bm,)
    return pl.pallas_call(
        _proj4_body,
        grid=grid,
        in_specs=[
            pl.BlockSpec((bm, k), lambda i: (i, 0)),
            pl.BlockSpec((k, n), lambda i: (0, 0)),
            pl.BlockSpec((1, n), lambda i: (0, 0)),
        ],
        out_specs=pl.BlockSpec((bm, n), lambda i: (i, 0)),
        out_shape=jax.ShapeDtypeStruct((m, n), jnp.float32),
    )(x, w, b.reshape(1, n))


def _pre_edge_body(x_ref, w_ref, b_ref, o_ref):
    o_ref[...] = jnp.maximum(
        jnp.dot(x_ref[...], w_ref[...], preferred_element_type=jnp.float32)
        + b_ref[...],
        0.0,
    ).astype(jnp.bfloat16)


def _pre_edge(x, w, b, bm):
    m, k = x.shape
    n = w.shape[1]
    grid = (m // bm,)
    return pl.pallas_call(
        _pre_edge_body,
        grid=grid,
        in_specs=[
            pl.BlockSpec((bm, k), lambda i: (i, 0)),
            pl.BlockSpec((k, n), lambda i: (0, 0)),
            pl.BlockSpec((1, n), lambda i: (0, 0)),
        ],
        out_specs=pl.BlockSpec((bm, n), lambda i: (i, 0)),
        out_shape=jax.ShapeDtypeStruct((m, n), jnp.bfloat16),
    )(x, w, b.reshape(1, n))


def _bilinear_body(t0_ref, t1_ref, t2_ref, t3_ref, u4_ref, w2_ref, b2r_ref,
                   r_ref, s_ref, o4_ref):
    h = r_ref.shape[0]
    t_refs = (t0_ref, t1_ref, t2_ref, t3_ref)
    outs = []
    for j in range(4):
        uj = u4_ref[:, j * h : (j + 1) * h]
        uj_b = uj.astype(jnp.bfloat16)
        g = jnp.dot(
            t_refs[j][...], w2_ref[...], preferred_element_type=jnp.float32
        )
        urep = jnp.dot(uj_b, r_ref[...], preferred_element_type=jnp.float32)
        prod = (g * urep).astype(jnp.bfloat16)
        outs.append(
            jnp.dot(prod, s_ref[...], preferred_element_type=jnp.float32)
            + jnp.dot(uj, b2r_ref[...], preferred_element_type=jnp.float32)
        )
    o4_ref[...] = jnp.concatenate(outs, axis=1)


def _bilinear(t, u4, we2, b2r, rmat, smat, br):
    e, eh = t.shape
    hh = we2.shape[1]
    h = rmat.shape[0]
    nblk = (e // 4) // br
    grid = (nblk,)

    def _tmap(j):
        return lambda i: (j * nblk + i, 0)

    return pl.pallas_call(
        _bilinear_body,
        grid=grid,
        in_specs=[
            pl.BlockSpec((br, eh), _tmap(0)),
            pl.BlockSpec((br, eh), _tmap(1)),
            pl.BlockSpec((br, eh), _tmap(2)),
            pl.BlockSpec((br, eh), _tmap(3)),
            pl.BlockSpec((br, 4 * h), lambda i: (i, 0)),
            pl.BlockSpec((eh, hh), lambda i: (0, 0)),
            pl.BlockSpec((h, h), lambda i: (0, 0)),
            pl.BlockSpec((h, hh), lambda i: (0, 0)),
            pl.BlockSpec((hh, h), lambda i: (0, 0)),
        ],
        out_specs=pl.BlockSpec((br, 4 * h), lambda i: (i, 0)),
        out_shape=jax.ShapeDtypeStruct((e // 4, 4 * h), jnp.float32),
    )(t, t, t, t, u4, we2, b2r, rmat, smat)


def _gru_body(p0_ref, p1_ref, bc_ref, hid_ref, wih_ref, whh_ref, bih_ref,
              bhh_ref, o_ref):
    h = bc_ref.shape[1]
    agg = p0_ref[...] + p1_ref[...] + bc_ref[...]
    hid = hid_ref[...]
    m = jnp.maximum(agg, 0.0)
    gi = (
        jnp.dot(m, wih_ref[...], preferred_element_type=jnp.float32)
        + bih_ref[...]
    )
    gh = (
        jnp.dot(hid, whh_ref[...], preferred_element_type=jnp.float32)
        + bhh_ref[...]
    )
    r = jax.nn.sigmoid(gi[:, :h] + gh[:, :h])
    z = jax.nn.sigmoid(gi[:, h : 2 * h] + gh[:, h : 2 * h])
    n = jnp.tanh(gi[:, 2 * h :] + r * gh[:, 2 * h :])
    o_ref[...] = (1.0 - z) * n + z * hid


def _gru(p0, p1, b_conv, hid, wih_t, whh_t, bih, bhh, bn):
    h = b_conv.shape[0]
    nn = hid.shape[0]
    grid = (nn // bn,)
    return pl.pallas_call(
        _gru_body,
        grid=grid,
        in_specs=[
            pl.BlockSpec((bn, h), lambda i: (i, 0)),
            pl.BlockSpec((bn, h), lambda i: (i, 0)),
            pl.BlockSpec((1, h), lambda i: (0, 0)),
            pl.BlockSpec((bn, h), lambda i: (i, 0)),
            pl.BlockSpec((h, 3 * h), lambda i: (0, 0)),
            pl.BlockSpec((h, 3 * h), lambda i: (0, 0)),
            pl.BlockSpec((1, 3 * h), lambda i: (0, 0)),
            pl.BlockSpec((1, 3 * h), lambda i: (0, 0)),
        ],
        out_specs=pl.BlockSpec((bn, h), lambda i: (i, 0)),
        out_shape=jax.ShapeDtypeStruct((nn, h), jnp.float32),
    )(
        p0,
        p1,
        b_conv.reshape(1, h),
        hid,
        wih_t,
        whh_t,
        bih.reshape(1, 3 * h),
        bhh.reshape(1, 3 * h),
    )


def _s2s_body(
    h_ref,
    wih0_ref, whh0_ref, bih0_ref, bhh0_ref,
    wih1_ref, whh1_ref, bih1_ref, bhh1_ref,
    wih2_ref, whh2_ref, bih2_ref, bhh2_ref,
    wd1_ref, bd1_ref, wd2_ref, bd2_ref,
    o_ref,
):
    hdim = h_ref.shape[1]
    hmat = h_ref[...]
    wih = [wih0_ref[...], wih1_ref[...], wih2_ref[...]]
    whh = [whh0_ref[...], whh1_ref[...], whh2_ref[...]]
    bih = [bih0_ref[...], bih1_ref[...], bih2_ref[...]]
    bhh = [bhh0_ref[...], bhh1_ref[...], bhh2_ref[...]]
    q_star = jnp.zeros((1, 2 * hdim), dtype=jnp.float32)
    hs = [jnp.zeros((1, hdim), dtype=jnp.float32) for _ in range(3)]
    cs = [jnp.zeros((1, hdim), dtype=jnp.float32) for _ in range(3)]
    for _ in range(6):
        x = q_star
        for l in range(3):
            g = (
                jnp.dot(x, wih[l], preferred_element_type=jnp.float32)
                + bih[l]
                + jnp.dot(hs[l], whh[l], preferred_element_type=jnp.float32)
                + bhh[l]
            )
            i = jax.nn.sigmoid(g[:, :hdim])
            f = jax.nn.sigmoid(g[:, hdim : 2 * hdim])
            gg = jnp.tanh(g[:, 2 * hdim : 3 * hdim])
            o = jax.nn.sigmoid(g[:, 3 * hdim :])
            cs[l] = f * cs[l] + i * gg
            hs[l] = o * jnp.tanh(cs[l])
            x = hs[l]
        q = x
        e = jnp.dot(hmat, q.reshape(hdim, 1), preferred_element_type=jnp.float32)
        emax = jnp.max(e)
        a = jnp.exp(e - emax)
        denom = jnp.sum(a)
        readout = jnp.sum(a * hmat, axis=0, keepdims=True) / denom
        q_star = jnp.concatenate([q, readout], axis=1)
    out = (
        jnp.dot(
            jnp.maximum(
                jnp.dot(q_star, wd1_ref[...], preferred_element_type=jnp.float32)
                + bd1_ref[...],
                0.0,
            ),
            wd2_ref[...],
            preferred_element_type=jnp.float32,
        )
        + bd2_ref[...]
    )
    o_ref[...] = out


def _set2set_decode(hmat, lstm_t, wd1, bd1, wd2, bd2):
    ph = wd1.shape[1]
    tasks = wd2.shape[1]
    args = [hmat]
    in_specs = [pl.BlockSpec(hmat.shape, lambda: (0, 0))]
    for (wih_t, whh_t, bih, bhh) in lstm_t:
        for a in (wih_t, whh_t, bih.reshape(1, -1), bhh.reshape(1, -1)):
            args.append(a)
            in_specs.append(pl.BlockSpec(a.shape, lambda: (0, 0)))
    for a in (wd1, bd1.reshape(1, ph), wd2, bd2.reshape(1, tasks)):
        args.append(a)
        in_specs.append(pl.BlockSpec(a.shape, lambda: (0, 0)))
    return pl.pallas_call(
        _s2s_body,
        in_specs=in_specs,
        out_specs=pl.BlockSpec((1, tasks), lambda: (0, 0)),
        out_shape=jax.ShapeDtypeStruct((1, tasks), jnp.float32),
    )(*args)


def _sc_gather(h, srcm):
    nn, d = h.shape
    nch = srcm.shape[0]
    etot = nch * _ECH
    per = nch // _NW
    rem = nch % _NW
    mesh = plsc.VectorSubcoreMesh(core_axis_name="c", subcore_axis_name="s")

    @functools.partial(
        pl.kernel,
        mesh=mesh,
        out_type=jax.ShapeDtypeStruct((etot, d), jnp.float32),
        scratch_types=[
            pltpu.VMEM((_K, _ECH), jnp.int32),
            pltpu.VMEM((_K * _ECH, d), jnp.float32),
            pltpu.SemaphoreType.DMA,
        ],
        compiler_params=pltpu.CompilerParams(use_tc_tiling_on_sc=False),
    )
    def k(h_hbm, srcm_hbm, out_hbm, idx2, rows_v, sem):
        c = lax.axis_index("c")
        s = lax.axis_index("s")
        wid = s * 2 + c
        base_row = wid * per

        def outer(row0, nrows):
            pltpu.sync_copy(srcm_hbm.at[pl.ds(row0, nrows)], idx2.at[pl.ds(0, nrows)])
            handles = []
            for j in range(nrows):
                handles.append(
                    pltpu.async_copy(
                        h_hbm.at[idx2.at[j]],
                        rows_v.at[pl.ds(j * _ECH, _ECH)],
                        sem,
                    )
                )
            for hd in handles:
                hd.wait()
            pltpu.sync_copy(
                rows_v.at[pl.ds(0, nrows * _ECH)],
                out_hbm.at[pl.ds(row0 * _ECH, nrows * _ECH)],
            )

        nfull, tail = divmod(per, _K)
        for o in range(nfull):
            outer(base_row + o * _K, _K)
        if tail:
            outer(base_row + nfull * _K, tail)
        if rem:
            @pl.when(wid < rem)
            def _():
                outer(_NW * per + wid, 1)

    return k(h, srcm)


def _sc_scatter(msg, dstm, zeros_nd):
    nch = dstm.shape[0]
    nn, d = zeros_nd.shape
    per = nch // _NW
    rem = nch % _NW
    rpt = nn // 16
    mesh = plsc.VectorSubcoreMesh(core_axis_name="c", subcore_axis_name="s")

    @functools.partial(
        pl.kernel,
        mesh=mesh,
        out_type=jax.ShapeDtypeStruct((2, nn, d), jnp.float32),
        scratch_types=[
            pltpu.VMEM((_K, _ECH), jnp.int32),
            pltpu.VMEM((_K * _ECH, d), jnp.float32),
            pltpu.VMEM_SHARED((nn, d), jnp.float32),
            pltpu.SemaphoreType.DMA,
        ],
        compiler_params=pltpu.CompilerParams(use_tc_tiling_on_sc=False),
    )
    def k(msg_hbm, dstm_hbm, zeros_hbm, out_hbm, idx2, rows_v, acc_sh, sem):
        c = lax.axis_index("c")
        s = lax.axis_index("s")
        wid = s * 2 + c
        base_row = wid * per
        msg_r = msg_hbm
        pltpu.sync_copy(
            zeros_hbm.at[pl.ds(s * rpt, rpt)], acc_sh.at[pl.ds(s * rpt, rpt)]
        )
        plsc.subcore_barrier()

        def outer(row0, nrows):
            pltpu.sync_copy(dstm_hbm.at[pl.ds(row0, nrows)], idx2.at[pl.ds(0, nrows)])
            pltpu.sync_copy(
                msg_r.at[pl.ds(row0 * _ECH, nrows * _ECH)],
                rows_v.at[pl.ds(0, nrows * _ECH)],
            )
            handles = []
            for j in range(nrows):
                handles.append(
                    pltpu.async_copy(
                        rows_v.at[pl.ds(j * _ECH, _ECH)],
                        acc_sh.at[idx2.at[j]],
                        sem,
                        add=True,
                    )
                )
            for hd in handles:
                hd.wait()

        nfull, tail = divmod(per, _K)
        for o in range(nfull):
            outer(base_row + o * _K, _K)
        if tail:
            outer(base_row + nfull * _K, tail)
        if rem:
            @pl.when(wid < rem)
            def _():
                outer(_NW * per + wid, 1)

        plsc.subcore_barrier()
        pltpu.sync_copy(
            acc_sh.at[pl.ds(s * rpt, rpt)], out_hbm.at[c, pl.ds(s * rpt, rpt)]
        )

    return k(msg, dstm, zeros_nd)


def kernel(node_feats, edge_feats, edge_index, W_proj, b_proj, We1, be1, We2,
           be2, b_conv, gWih, gWhh, gbih, gbhh, lWih0, lWhh0, lbih0, lbhh0,
           lWih1, lWhh1, lbih1, lbhh1, lWih2, lWhh2, lbih2, lbhh2, Wd1, bd1,
           Wd2, bd2):
    nn = node_feats.shape[0]
    etot = edge_feats.shape[0]
    h = W_proj.shape[1]
    hh = We2.shape[1]
    perm_si = edge_index[:, :].reshape(2, 4, etot // 4)
    src_p = perm_si[0].T.reshape(etot)
    dst_p = perm_si[1].T.reshape(etot)
    srcm = src_p.reshape(etot // _ECH, _ECH)
    dstm = dst_p.reshape(etot // _ECH, _ECH)

    cols = jnp.arange(hh, dtype=jnp.int32)
    rmat = (cols[None, :] // h == jnp.arange(h, dtype=jnp.int32)[:, None]).astype(
        jnp.bfloat16
    )
    smat = (cols[:, None] % h == jnp.arange(h, dtype=jnp.int32)[None, :]).astype(
        jnp.bfloat16
    )
    zeros_nd = jnp.zeros((nn, h), dtype=jnp.float32)
    we2_b = We2.astype(jnp.bfloat16)
    b2r = be2.reshape(h, h)

    t = _pre_edge(edge_feats, We1, be1, 2000)
    hcur = _relu_proj4(node_feats, W_proj, b_proj, 10000)
    hidden = hcur

    wih_t = gWih.T
    whh_t = gWhh.T

    for _ in range(5):
        u = _sc_gather(hcur, srcm)
        u4 = jnp.reshape(u, (etot // 4, 4 * h))
        msg4 = _bilinear(t, u4, we2_b, b2r, rmat, smat, 400)
        partials = _sc_scatter(jnp.reshape(msg4, (etot, h)), dstm, zeros_nd)
        hidden = _gru(partials[0], partials[1], b_conv, hidden, wih_t, whh_t,
                      gbih, gbhh, 10000)
        hcur = hidden

    lstm_t = [
        (lWih0.T, lWhh0.T, lbih0, lbhh0),
        (lWih1.T, lWhh1.T, lbih1, lbhh1),
        (lWih2.T, lWhh2.T, lbih2, lbhh2),
    ]
    return _set2set_decode(hcur, lstm_t, Wd1, bd1, Wd2, bd2)


# speedup vs baseline: 1.2010x; 1.0136x over previous
import functools

import jax
import jax.numpy as jnp
from jax import lax
from jax.experimental import pallas as pl
from jax.experimental.pallas import tpu as pltpu
from jax.experimental.pallas import tpu_sc as plsc

_NW = 32
_ECH = 128
_K = 8


def _pre_edge_body(x_ref, w_ref, b_ref, o_ref):
    o_ref[...] = jnp.maximum(
        jnp.dot(x_ref[...], w_ref[...], preferred_element_type=jnp.float32)
        + b_ref[...],
        0.0,
    ).astype(jnp.bfloat16)


def _pre_edge(x, w, b, bm):
    m, k = x.shape
    n = w.shape[1]
    grid = (m // bm,)
    return pl.pallas_call(
        _pre_edge_body,
        grid=grid,
        in_specs=[
            pl.BlockSpec((bm, k), lambda i: (i, 0)),
            pl.BlockSpec((k, n), lambda i: (0, 0)),
            pl.BlockSpec((1, n), lambda i: (0, 0)),
        ],
        out_specs=pl.BlockSpec((bm, n), lambda i: (i, 0)),
        out_shape=jax.ShapeDtypeStruct((m, n), jnp.bfloat16),
    )(x, w, b.reshape(1, n))


def _pre_node4_body(x4_ref, w_ref, b_ref, o4_ref):
    k = w_ref.shape[0]
    outs = []
    for j in range(4):
        xj = x4_ref[:, j * k : (j + 1) * k]
        outs.append(
            jnp.maximum(
                jnp.dot(xj, w_ref[...], preferred_element_type=jnp.float32)
                + b_ref[...],
                0.0,
            )
        )
    o4_ref[...] = jnp.concatenate(outs, axis=1)


def _pre_node4(x4, w, b):
    nn4 = x4.shape[0]
    k, n = w.shape
    return pl.pallas_call(
        _pre_node4_body,
        in_specs=[
            pl.BlockSpec(x4.shape, lambda: (0, 0)),
            pl.BlockSpec((k, n), lambda: (0, 0)),
            pl.BlockSpec((1, n), lambda: (0, 0)),
        ],
        out_specs=pl.BlockSpec((nn4, 4 * n), lambda: (0, 0)),
        out_shape=jax.ShapeDtypeStruct((nn4, 4 * n), jnp.float32),
    )(x4, w, b.reshape(1, n))


def _bilinear_body(t0_ref, t1_ref, t2_ref, t3_ref, u4_ref, w2_ref, b2r_ref,
                   r_ref, s_ref, o4_ref):
    h = r_ref.shape[0]
    br = u4_ref.shape[0]
    t_all = jnp.concatenate(
        [t0_ref[...], t1_ref[...], t2_ref[...], t3_ref[...]], axis=0
    )
    u_all = jnp.concatenate(
        [u4_ref[:, j * h : (j + 1) * h] for j in range(4)], axis=0
    )
    u_b = u_all.astype(jnp.bfloat16)
    g = jnp.dot(t_all, w2_ref[...], preferred_element_type=jnp.float32)
    urep = jnp.dot(u_b, r_ref[...], preferred_element_type=jnp.float32)
    prod = (g * urep).astype(jnp.bfloat16)
    msg = jnp.dot(
        prod, s_ref[...], preferred_element_type=jnp.float32
    ) + jnp.dot(u_all, b2r_ref[...], preferred_element_type=jnp.float32)
    o4_ref[...] = jnp.concatenate(
        [msg[j * br : (j + 1) * br] for j in range(4)], axis=1
    )


def _bilinear(t, u4, we2, b2r, rmat, smat, br, half_ofs, grp_stride):
    eh = t.shape[1]
    hh = we2.shape[1]
    h = rmat.shape[0]
    rows = u4.shape[0]
    nblk = rows // br
    grid = (nblk,)

    def _tmap(j):
        return lambda i: (j * grp_stride + half_ofs + i, 0)

    return pl.pallas_call(
        _bilinear_body,
        grid=grid,
        in_specs=[
            pl.BlockSpec((br, eh), _tmap(0)),
            pl.BlockSpec((br, eh), _tmap(1)),
            pl.BlockSpec((br, eh), _tmap(2)),
            pl.BlockSpec((br, eh), _tmap(3)),
            pl.BlockSpec((br, 4 * h), lambda i: (i, 0)),
            pl.BlockSpec((eh, hh), lambda i: (0, 0)),
            pl.BlockSpec((h, h), lambda i: (0, 0)),
            pl.BlockSpec((h, hh), lambda i: (0, 0)),
            pl.BlockSpec((hh, h), lambda i: (0, 0)),
        ],
        out_specs=pl.BlockSpec((br, 4 * h), lambda i: (i, 0)),
        out_shape=jax.ShapeDtypeStruct((rows, 4 * h), jnp.float32),
    )(t, t, t, t, u4, we2, b2r, rmat, smat)


def _gru4_body(pa0_ref, pa1_ref, pb0_ref, pb1_ref, bc_ref, hid4_ref, wih_ref,
               whh_ref, bih_ref, bhh_ref, o4_ref):
    h = bc_ref.shape[1]
    outs = []
    for j in range(4):
        sl = slice(j * h, (j + 1) * h)
        agg = (
            pa0_ref[:, sl]
            + pa1_ref[:, sl]
            + pb0_ref[:, sl]
            + pb1_ref[:, sl]
            + bc_ref[...]
        )
        hid = hid4_ref[:, sl]
        m = jnp.maximum(agg, 0.0)
        gi = (
            jnp.dot(m, wih_ref[...], preferred_element_type=jnp.float32)
            + bih_ref[...]
        )
        gh = (
            jnp.dot(hid, whh_ref[...], preferred_element_type=jnp.float32)
            + bhh_ref[...]
        )
        r = jax.nn.sigmoid(gi[:, :h] + gh[:, :h])
        z = jax.nn.sigmoid(gi[:, h : 2 * h] + gh[:, h : 2 * h])
        n = jnp.tanh(gi[:, 2 * h :] + r * gh[:, 2 * h :])
        outs.append((1.0 - z) * n + z * hid)
    o4_ref[...] = jnp.concatenate(outs, axis=1)


def _gru4(pa0, pa1, pb0, pb1, b_conv, hid4, wih_t, whh_t, bih, bhh):
    h = b_conv.shape[0]
    nn4 = hid4.shape[0]
    full = lambda a: pl.BlockSpec(a.shape, lambda: (0, 0))
    args = [
        pa0, pa1, pb0, pb1,
        b_conv.reshape(1, h), hid4, wih_t, whh_t,
        bih.reshape(1, 3 * h), bhh.reshape(1, 3 * h),
    ]
    return pl.pallas_call(
        _gru4_body,
        in_specs=[full(a) for a in args],
        out_specs=pl.BlockSpec((nn4, 4 * h), lambda: (0, 0)),
        out_shape=jax.ShapeDtypeStruct((nn4, 4 * h), jnp.float32),
    )(*args)


def _s2s_body(
    h4_ref,
    wih0_ref, whh0_ref, bih0_ref, bhh0_ref,
    wih1_ref, whh1_ref, bih1_ref, bhh1_ref,
    wih2_ref, whh2_ref, bih2_ref, bhh2_ref,
    wd1_ref, bd1_ref, wd2_ref, bd2_ref,
    o_ref,
):
    hdim = h4_ref.shape[1] // 4
    hj = [h4_ref[:, j * hdim : (j + 1) * hdim] for j in range(4)]
    wih = [wih0_ref[...], wih1_ref[...], wih2_ref[...]]
    whh = [whh0_ref[...], whh1_ref[...], whh2_ref[...]]
    bih = [bih0_ref[...], bih1_ref[...], bih2_ref[...]]
    bhh = [bhh0_ref[...], bhh1_ref[...], bhh2_ref[...]]
    q_star = jnp.zeros((1, 2 * hdim), dtype=jnp.float32)
    hs = [jnp.zeros((1, hdim), dtype=jnp.float32) for _ in range(3)]
    cs = [jnp.zeros((1, hdim), dtype=jnp.float32) for _ in range(3)]
    for _ in range(6):
        x = q_star
        for l in range(3):
            g = (
                jnp.dot(x, wih[l], preferred_element_type=jnp.float32)
                + bih[l]
                + jnp.dot(hs[l], whh[l], preferred_element_type=jnp.float32)
                + bhh[l]
            )
            i = jax.nn.sigmoid(g[:, :hdim])
            f = jax.nn.sigmoid(g[:, hdim : 2 * hdim])
            gg = jnp.tanh(g[:, 2 * hdim : 3 * hdim])
            o = jax.nn.sigmoid(g[:, 3 * hdim :])
            cs[l] = f * cs[l] + i * gg
            hs[l] = o * jnp.tanh(cs[l])
            x = hs[l]
        q = x
        qt = q.reshape(hdim, 1)
        ej = [
            jnp.dot(hj[j], qt, preferred_element_type=jnp.float32)
            for j in range(4)
        ]
        emax = jnp.maximum(
            jnp.maximum(jnp.max(ej[0]), jnp.max(ej[1])),
            jnp.maximum(jnp.max(ej[2]), jnp.max(ej[3])),
        )
        aj = [jnp.exp(e - emax) for e in ej]
        denom = sum(jnp.sum(a) for a in aj)
        readout = (
            sum(
                jnp.sum(aj[j] * hj[j], axis=0, keepdims=True)
                for j in range(4)
            )
            / denom
        )
        q_star = jnp.concatenate([q, readout], axis=1)
    out = (
        jnp.dot(
            jnp.maximum(
                jnp.dot(q_star, wd1_ref[...], preferred_element_type=jnp.float32)
                + bd1_ref[...],
                0.0,
            ),
            wd2_ref[...],
            preferred_element_type=jnp.float32,
        )
        + bd2_ref[...]
    )
    o_ref[...] = out


def _set2set_decode(h4, lstm_t, wd1, bd1, wd2, bd2):
    ph = wd1.shape[1]
    tasks = wd2.shape[1]
    args = [h4]
    for (wih_t, whh_t, bih, bhh) in lstm_t:
        args.extend((wih_t, whh_t, bih.reshape(1, -1), bhh.reshape(1, -1)))
    args.extend((wd1, bd1.reshape(1, ph), wd2, bd2.reshape(1, tasks)))
    in_specs = [pl.BlockSpec(a.shape, lambda: (0, 0)) for a in args]
    return pl.pallas_call(
        _s2s_body,
        in_specs=in_specs,
        out_specs=pl.BlockSpec((1, tasks), lambda: (0, 0)),
        out_shape=jax.ShapeDtypeStruct((1, tasks), jnp.float32),
    )(*args)


def _sc_gather(h, srcm):
    nn, d = h.shape
    nch = srcm.shape[0]
    etot = nch * _ECH
    per = nch // _NW
    rem = nch % _NW
    mesh = plsc.VectorSubcoreMesh(core_axis_name="c", subcore_axis_name="s")

    @functools.partial(
        pl.kernel,
        mesh=mesh,
        out_type=jax.ShapeDtypeStruct((etot, d), jnp.float32),
        scratch_types=[
            pltpu.VMEM((_K, _ECH), jnp.int32),
            pltpu.VMEM((_K * _ECH, d), jnp.float32),
            pltpu.SemaphoreType.DMA,
        ],
        compiler_params=pltpu.CompilerParams(use_tc_tiling_on_sc=False),
    )
    def k(h_hbm, srcm_hbm, out_hbm, idx2, rows_v, sem):
        c = lax.axis_index("c")
        s = lax.axis_index("s")
        wid = s * 2 + c
        base_row = wid * per

        def outer(row0, nrows):
            pltpu.sync_copy(srcm_hbm.at[pl.ds(row0, nrows)], idx2.at[pl.ds(0, nrows)])
            handles = []
            for j in range(nrows):
                handles.append(
                    pltpu.async_copy(
                        h_hbm.at[idx2.at[j]],
                        rows_v.at[pl.ds(j * _ECH, _ECH)],
                        sem,
                    )
                )
            for hd in handles:
                hd.wait()
            pltpu.sync_copy(
                rows_v.at[pl.ds(0, nrows * _ECH)],
                out_hbm.at[pl.ds(row0 * _ECH, nrows * _ECH)],
            )

        nfull, tail = divmod(per, _K)
        for o in range(nfull):
            outer(base_row + o * _K, _K)
        if tail:
            outer(base_row + nfull * _K, tail)
        if rem:
            @pl.when(wid < rem)
            def _():
                outer(_NW * per + wid, 1)

    return k(h, srcm)


def _sc_scatter(msg, dstm, zeros_nd):
    nch = dstm.shape[0]
    nn, d = zeros_nd.shape
    per = nch // _NW
    rem = nch % _NW
    rpt = nn // 16
    mesh = plsc.VectorSubcoreMesh(core_axis_name="c", subcore_axis_name="s")

    @functools.partial(
        pl.kernel,
        mesh=mesh,
        out_type=jax.ShapeDtypeStruct((2, nn, d), jnp.float32),
        scratch_types=[
            pltpu.VMEM((_K, _ECH), jnp.int32),
            pltpu.VMEM((_K * _ECH, d), jnp.float32),
            pltpu.VMEM_SHARED((nn, d), jnp.float32),
            pltpu.SemaphoreType.DMA,
        ],
        compiler_params=pltpu.CompilerParams(use_tc_tiling_on_sc=False),
    )
    def k(msg_hbm, dstm_hbm, zeros_hbm, out_hbm, idx2, rows_v, acc_sh, sem):
        c = lax.axis_index("c")
        s = lax.axis_index("s")
        wid = s * 2 + c
        base_row = wid * per
        pltpu.sync_copy(
            zeros_hbm.at[pl.ds(s * rpt, rpt)], acc_sh.at[pl.ds(s * rpt, rpt)]
        )
        plsc.subcore_barrier()

        def outer(row0, nrows):
            pltpu.sync_copy(dstm_hbm.at[pl.ds(row0, nrows)], idx2.at[pl.ds(0, nrows)])
            pltpu.sync_copy(
                msg_hbm.at[pl.ds(row0 * _ECH, nrows * _ECH)],
                rows_v.at[pl.ds(0, nrows * _ECH)],
            )
            handles = []
            for j in range(nrows):
                handles.append(
                    pltpu.async_copy(
                        rows_v.at[pl.ds(j * _ECH, _ECH)],
                        acc_sh.at[idx2.at[j]],
                        sem,
                        add=True,
                    )
                )
            for hd in handles:
                hd.wait()

        nfull, tail = divmod(per, _K)
        for o in range(nfull):
            outer(base_row + o * _K, _K)
        if tail:
            outer(base_row + nfull * _K, tail)
        if rem:
            @pl.when(wid < rem)
            def _():
                outer(_NW * per + wid, 1)

        plsc.subcore_barrier()
        pltpu.sync_copy(
            acc_sh.at[pl.ds(s * rpt, rpt)], out_hbm.at[c, pl.ds(s * rpt, rpt)]
        )

    return k(msg, dstm, zeros_nd)


def kernel(node_feats, edge_feats, edge_index, W_proj, b_proj, We1, be1, We2,
           be2, b_conv, gWih, gWhh, gbih, gbhh, lWih0, lWhh0, lbih0, lbhh0,
           lWih1, lWhh1, lbih1, lbhh1, lWih2, lWhh2, lbih2, lbhh2, Wd1, bd1,
           Wd2, bd2):
    nn = node_feats.shape[0]
    etot = edge_feats.shape[0]
    h = W_proj.shape[1]
    hh = We2.shape[1]
    eq = etot // 4
    half = etot // 2
    nch_half = half // _ECH

    perm_si = edge_index.reshape(2, 4, eq)
    src_p = perm_si[0].T.reshape(etot)
    dst_p = perm_si[1].T.reshape(etot)
    srcm = src_p.reshape(etot // _ECH, _ECH)
    dstm = dst_p.reshape(etot // _ECH, _ECH)

    cols = jnp.arange(hh, dtype=jnp.int32)
    rmat = (cols[None, :] // h == jnp.arange(h, dtype=jnp.int32)[:, None]).astype(
        jnp.bfloat16
    )
    smat = (cols[:, None] % h == jnp.arange(h, dtype=jnp.int32)[None, :]).astype(
        jnp.bfloat16
    )
    zeros_nd = jnp.zeros((nn, h), dtype=jnp.float32)
    we2_b = We2.astype(jnp.bfloat16)
    b2r = be2.reshape(h, h)

    t = _pre_edge(edge_feats, We1, be1, 2000)
    h4 = _pre_node4(node_feats.reshape(nn // 4, 512), W_proj, b_proj)
    hid4 = h4

    wih_t = gWih.T
    whh_t = gWhh.T
    grp_stride = eq // 400
    half_blk = (half // 4) // 400

    for _ in range(5):
        hflat = jnp.reshape(h4, (nn, h))
        uA = _sc_gather(hflat, srcm[:nch_half])
        uB = _sc_gather(hflat, srcm[nch_half:])
        u4A = jnp.reshape(uA, (half // 4, 4 * h))
        u4B = jnp.reshape(uB, (half // 4, 4 * h))
        msg4A = _bilinear(t, u4A, we2_b, b2r, rmat, smat, 400, 0, grp_stride)
        msg4B = _bilinear(
            t, u4B, we2_b, b2r, rmat, smat, 400, half_blk, grp_stride
        )
        partA = _sc_scatter(jnp.reshape(msg4A, (half, h)), dstm[:nch_half], zeros_nd)
        partB = _sc_scatter(jnp.reshape(msg4B, (half, h)), dstm[nch_half:], zeros_nd)
        pa0 = jnp.reshape(partA[0], (nn // 4, 4 * h))
        pa1 = jnp.reshape(partA[1], (nn // 4, 4 * h))
        pb0 = jnp.reshape(partB[0], (nn // 4, 4 * h))
        pb1 = jnp.reshape(partB[1], (nn // 4, 4 * h))
        hid4 = _gru4(pa0, pa1, pb0, pb1, b_conv, hid4, wih_t, whh_t, gbih, gbhh)
        h4 = hid4

    lstm_t = [
        (lWih0.T, lWhh0.T, lbih0, lbhh0),
        (lWih1.T, lWhh1.T, lbih1, lbhh1),
        (lWih2.T, lWhh2.T, lbih2, lbhh2),
    ]
    return _set2set_decode(h4, lstm_t, Wd1, bd1, Wd2, bd2)
